# baseline (speedup 1.0000x reference)
import functools

import jax
import jax.numpy as jnp
from jax import lax
from jax.experimental import pallas as pl
from jax.experimental.pallas import tpu as pltpu
from jax.experimental.pallas import tpu_sc as plsc

N = 10000
E = 320000
D_IN = 128
D_HID = 128
D_OUT = 16

NC = 2
NS = 16
NW = NC * NS

NP = 10240
TRASH = 10000
EP = 327680
EC = 64
IDX_ROWS = EP // EC
ROWS_PER_W = IDX_ROWS // NW
IB = 16
ZCHUNK = NP // NS


def _mesh():
    return plsc.VectorSubcoreMesh(
        core_axis_name="c", subcore_axis_name="s", num_cores=NC, num_subcores=NS
    )


def _deg_body(dst_hbm, out_hbm, idx_v, ones_v, zeros_v, sem, acc_sh):
    c = lax.axis_index("c")
    s = lax.axis_index("s")
    wid = c * NS + s

    def _z(i, _):
        zeros_v[pl.ds(i * 16, 16)] = jnp.zeros((16,), jnp.float32)
        return _
    lax.fori_loop(0, ZCHUNK // 16, _z, None)
    for k in range(EC // 16):
        ones_v[pl.ds(k * 16, 16)] = jnp.ones((16,), jnp.float32)

    pltpu.sync_copy(zeros_v, acc_sh.at[pl.ds(s * ZCHUNK, ZCHUNK)])
    plsc.subcore_barrier()

    def _blk(b, _):
        pltpu.sync_copy(dst_hbm.at[pl.ds(wid * ROWS_PER_W + b * IB, IB)], idx_v)
        for j in range(IB):
            pltpu.async_copy(ones_v, acc_sh.at[idx_v.at[j]], sem, add=True)
        for j in range(IB):
            pltpu.make_async_copy(ones_v, acc_sh.at[idx_v.at[0]], sem).wait()
        return _
    lax.fori_loop(0, ROWS_PER_W // IB, _blk, None)

    plsc.subcore_barrier()
    pltpu.sync_copy(
        acc_sh.at[pl.ds(s * ZCHUNK, ZCHUNK)],
        out_hbm.at[c, pl.ds(s * ZCHUNK, ZCHUNK)],
    )


def _make_deg_kernel():
    return pl.kernel(
        _deg_body,
        out_type=jax.ShapeDtypeStruct((NC, NP), jnp.float32),
        mesh=_mesh(),
        scratch_types=[
            pltpu.VMEM((IB, EC), jnp.int32),
            pltpu.VMEM((EC,), jnp.float32),
            pltpu.VMEM((ZCHUNK,), jnp.float32),
            pltpu.SemaphoreType.DMA,
            pltpu.VMEM_SHARED((NP,), jnp.float32),
        ],
    )


def _agg_body(h_hbm, src_hbm, dst_hbm, out_hbm,
              src_v, dst_v, rows_v, *rest, dfeat, nslot, ahead, ec, ib,
              rows_per_w):
    sg = rest[:nslot]
    ss = rest[nslot:2 * nslot]
    acc_sh = rest[2 * nslot]
    c = lax.axis_index("c")
    s = lax.axis_index("s")
    wid = c * NS + s
    nvec = dfeat // 16

    def _z(r, _):
        for k in range(nvec):
            rows_v[0, r, pl.ds(k * 16, 16)] = jnp.zeros((16,), jnp.float32)
        return _
    lax.fori_loop(0, ec, _z, None)
    for i in range(ZCHUNK // ec):
        pltpu.sync_copy(rows_v.at[0],
                        acc_sh.at[pl.ds(s * ZCHUNK + i * ec, ec)])
    plsc.subcore_barrier()

    def _wait_scatter(slot):
        pltpu.make_async_copy(
            rows_v.at[slot], acc_sh.at[dst_v.at[0]], ss[slot]).wait()

    def _blk(b, _):
        @pl.when(b > 0)
        def _():
            for slot in range(nslot):
                _wait_scatter(slot)
        base = wid * rows_per_w + b * ib
        pltpu.sync_copy(src_hbm.at[pl.ds(base, ib)], src_v)
        pltpu.sync_copy(dst_hbm.at[pl.ds(base, ib)], dst_v)

        def _fire(j, slot):
            pltpu.async_copy(h_hbm.at[src_v.at[j]], rows_v.at[slot], sg[slot])

        for k in range(ahead):
            _fire(k, k % nslot)
        for j in range(ib):
            sj = j % nslot
            if j + ahead < ib:
                s2 = (j + ahead) % nslot
                if j + ahead >= nslot:
                    _wait_scatter(s2)
                _fire(j + ahead, s2)
            pltpu.make_async_copy(
                h_hbm.at[src_v.at[j]], rows_v.at[sj], sg[sj]).wait()
            pltpu.async_copy(rows_v.at[sj], acc_sh.at[dst_v.at[j]], ss[sj],
                             add=True)
        return _
    lax.fori_loop(0, rows_per_w // ib, _blk, None)
    for slot in range(nslot):
        _wait_scatter(slot)

    plsc.subcore_barrier()
    for i in range(ZCHUNK // ec):
        pltpu.sync_copy(acc_sh.at[pl.ds(s * ZCHUNK + i * ec, ec)],
                        out_hbm.at[c, pl.ds(s * ZCHUNK + i * ec, ec)])


def _make_agg_kernel(dfeat):
    if dfeat == 128:
        nslot, ahead, ec, ib = 4, 2, 64, 16
    else:
        nslot, ahead, ec, ib = 6, 3, 128, 16
    rows_per_w = EP // ec // NW
    assert EP % (ec * NW) == 0 and rows_per_w % ib == 0 and ZCHUNK % ec == 0
    return pl.kernel(
        functools.partial(_agg_body, dfeat=dfeat, nslot=nslot, ahead=ahead,
                          ec=ec, ib=ib, rows_per_w=rows_per_w),
        out_type=jax.ShapeDtypeStruct((NC, NP, dfeat), jnp.float32),
        mesh=_mesh(),
        scratch_types=[
            pltpu.VMEM((ib, ec), jnp.int32),
            pltpu.VMEM((ib, ec), jnp.int32),
            pltpu.VMEM((nslot, ec, dfeat), jnp.float32),
            *([pltpu.SemaphoreType.DMA] * (2 * nslot)),
            pltpu.VMEM_SHARED((NP, dfeat), jnp.float32),
        ],
        compiler_params=pltpu.CompilerParams(
            use_tc_tiling_on_sc=False if dfeat < 128 else None),
    )


def _tc1_body(x_ref, w1_ref, deg_ref, dis_ref, hp_ref):
    deg = 1.0 + deg_ref[0] + deg_ref[1]
    dis = lax.rsqrt(deg)
    dis_ref[...] = dis
    h = jnp.dot(x_ref[...], w1_ref[...], preferred_element_type=jnp.float32)
    hp_ref[...] = h * dis[:N]


def _tc2_body(p_ref, hp_ref, dis_ref, b1_ref, w2_ref, h2p_ref):
    d = dis_ref[...][:N]
    agg = p_ref[0, pl.ds(0, N), :] + p_ref[1, pl.ds(0, N), :] + hp_ref[...]
    z = jnp.maximum(agg * d + b1_ref[...], 0.0)
    h2 = jnp.dot(z, w2_ref[...], preferred_element_type=jnp.float32)
    h2p_ref[...] = h2 * d


def _tc3_body(q_ref, h2p_ref, dis_ref, b2_ref, out_ref):
    d = dis_ref[...][:N]
    logits = (q_ref[0, pl.ds(0, N), :] + q_ref[1, pl.ds(0, N), :]
              + h2p_ref[...]) * d + b2_ref[...]
    m = jnp.max(logits, axis=1, keepdims=True)
    lse = jnp.log(jnp.sum(jnp.exp(logits - m), axis=1, keepdims=True)) + m
    out_ref[...] = logits - lse


def _tc_call(body, out_shapes):
    return pl.pallas_call(body, out_shape=out_shapes)


@jax.jit
def kernel(x, edge_index, W1, b1, W2, b2):
    src = edge_index[0]
    dst = edge_index[1]
    pad = EP - E
    pad_i = jnp.arange(pad, dtype=jnp.int32)
    src2d = jnp.concatenate(
        [src, pad_i % N]).reshape(IDX_ROWS, EC)
    dst2d = jnp.concatenate(
        [dst, TRASH + pad_i % (NP - TRASH)]).reshape(IDX_ROWS, EC)

    deg_part = _make_deg_kernel()(dst2d)

    dis, hp = _tc_call(_tc1_body, (
        jax.ShapeDtypeStruct((NP, 1), jnp.float32),
        jax.ShapeDtypeStruct((N, D_HID), jnp.float32),
    ))(x, W1, deg_part.reshape(NC, NP, 1))

    p = _make_agg_kernel(D_HID)(hp, src2d, dst2d)
    src128 = src2d.reshape(EP // 128, 128)
    dst128 = dst2d.reshape(EP // 128, 128)

    h2p = _tc_call(_tc2_body, jax.ShapeDtypeStruct((N, D_OUT), jnp.float32))(
        p, hp, dis, b1.reshape(1, D_HID), W2)

    q = _make_agg_kernel(D_OUT)(h2p, src128, dst128)

    out = _tc_call(_tc3_body, jax.ShapeDtypeStruct((N, D_OUT), jnp.float32))(
        q, h2p, dis, b2.reshape(1, D_OUT))
    return out


# speedup vs baseline: 1.0886x; 1.0886x over previous
import functools

import jax
import jax.numpy as jnp
from jax import lax
from jax.experimental import pallas as pl
from jax.experimental.pallas import tpu as pltpu
from jax.experimental.pallas import tpu_sc as plsc

N = 10000
E = 320000
D_IN = 128
D_HID = 128
D_OUT = 16

NC = 2
NS = 16
NW = NC * NS

NP = 10240
TRASH = 10000
EP = 327680
EC = 64
IDX_ROWS = EP // EC
ROWS_PER_W = IDX_ROWS // NW
IB = 16
ZCHUNK = NP // NS


def _mesh():
    return plsc.VectorSubcoreMesh(
        core_axis_name="c", subcore_axis_name="s", num_cores=NC, num_subcores=NS
    )


def _deg_body(dst_hbm, out_hbm, idx_v, ones_v, zeros_v, sem, acc_sh):
    c = lax.axis_index("c")
    s = lax.axis_index("s")
    wid = c * NS + s

    def _z(i, _):
        zeros_v[pl.ds(i * 16, 16)] = jnp.zeros((16,), jnp.float32)
        return _
    lax.fori_loop(0, ZCHUNK // 16, _z, None)
    for k in range(EC // 16):
        ones_v[pl.ds(k * 16, 16)] = jnp.ones((16,), jnp.float32)

    pltpu.sync_copy(zeros_v, acc_sh.at[pl.ds(s * ZCHUNK, ZCHUNK)])
    plsc.subcore_barrier()

    def _blk(b, _):
        pltpu.sync_copy(dst_hbm.at[pl.ds(wid * ROWS_PER_W + b * IB, IB)], idx_v)
        for j in range(IB):
            pltpu.async_copy(ones_v, acc_sh.at[idx_v.at[j]], sem, add=True)
        for j in range(IB):
            pltpu.make_async_copy(ones_v, acc_sh.at[idx_v.at[0]], sem).wait()
        return _
    lax.fori_loop(0, ROWS_PER_W // IB, _blk, None)

    plsc.subcore_barrier()
    pltpu.sync_copy(
        acc_sh.at[pl.ds(s * ZCHUNK, ZCHUNK)],
        out_hbm.at[c, pl.ds(s * ZCHUNK, ZCHUNK)],
    )


def _make_deg_kernel():
    return pl.kernel(
        _deg_body,
        out_type=jax.ShapeDtypeStruct((NC, NP), jnp.float32),
        mesh=_mesh(),
        scratch_types=[
            pltpu.VMEM((IB, EC), jnp.int32),
            pltpu.VMEM((EC,), jnp.float32),
            pltpu.VMEM((ZCHUNK,), jnp.float32),
            pltpu.SemaphoreType.DMA,
            pltpu.VMEM_SHARED((NP,), jnp.float32),
        ],
    )


def _agg_body(h_hbm, src_hbm, dst_hbm, out_hbm,
              src_v, dst_v, rows_v, *rest, dfeat, nslot, ahead, ec, ib,
              rows_per_w, dtype):
    sg = rest[:nslot]
    ss = rest[nslot:2 * nslot]
    acc_sh = rest[2 * nslot]
    c = lax.axis_index("c")
    s = lax.axis_index("s")
    wid = c * NS + s
    lanes = 16 if dtype == jnp.float32 else 32
    nvec = dfeat // lanes

    def _z(r, _):
        for k in range(nvec):
            rows_v[0, r, pl.ds(k * lanes, lanes)] = jnp.zeros((lanes,), dtype)
        return _
    lax.fori_loop(0, ec, _z, None)
    for i in range(ZCHUNK // ec):
        pltpu.sync_copy(rows_v.at[0],
                        acc_sh.at[pl.ds(s * ZCHUNK + i * ec, ec)])
    plsc.subcore_barrier()

    def _wait_scatter(slot):
        pltpu.make_async_copy(
            rows_v.at[slot], acc_sh.at[dst_v.at[0]], ss[slot]).wait()

    def _blk(b, _):
        @pl.when(b > 0)
        def _():
            for slot in range(nslot):
                _wait_scatter(slot)
        base = wid * rows_per_w + b * ib
        pltpu.sync_copy(src_hbm.at[pl.ds(base, ib)], src_v)
        pltpu.sync_copy(dst_hbm.at[pl.ds(base, ib)], dst_v)

        def _fire(j, slot):
            pltpu.async_copy(h_hbm.at[src_v.at[j]], rows_v.at[slot], sg[slot])

        for k in range(ahead):
            _fire(k, k % nslot)
        for j in range(ib):
            sj = j % nslot
            if j + ahead < ib:
                s2 = (j + ahead) % nslot
                if j + ahead >= nslot:
                    _wait_scatter(s2)
                _fire(j + ahead, s2)
            pltpu.make_async_copy(
                h_hbm.at[src_v.at[j]], rows_v.at[sj], sg[sj]).wait()
            pltpu.async_copy(rows_v.at[sj], acc_sh.at[dst_v.at[j]], ss[sj],
                             add=True)
        return _
    lax.fori_loop(0, rows_per_w // ib, _blk, None)
    for slot in range(nslot):
        _wait_scatter(slot)

    plsc.subcore_barrier()
    for i in range(ZCHUNK // ec):
        pltpu.sync_copy(acc_sh.at[pl.ds(s * ZCHUNK + i * ec, ec)],
                        out_hbm.at[c, pl.ds(s * ZCHUNK + i * ec, ec)])


def _make_agg_kernel(dfeat):
    if dfeat == 128:
        nslot, ahead, ec, ib = 6, 3, 64, 16
        dtype = jnp.bfloat16
    else:
        nslot, ahead, ec, ib = 6, 3, 128, 16
        dtype = jnp.float32
    rows_per_w = EP // ec // NW
    assert EP % (ec * NW) == 0 and rows_per_w % ib == 0 and ZCHUNK % ec == 0
    return pl.kernel(
        functools.partial(_agg_body, dfeat=dfeat, nslot=nslot, ahead=ahead,
                          ec=ec, ib=ib, rows_per_w=rows_per_w, dtype=dtype),
        out_type=jax.ShapeDtypeStruct((NC, NP, dfeat), dtype),
        mesh=_mesh(),
        scratch_types=[
            pltpu.VMEM((ib, ec), jnp.int32),
            pltpu.VMEM((ib, ec), jnp.int32),
            pltpu.VMEM((nslot, ec, dfeat), dtype),
            *([pltpu.SemaphoreType.DMA] * (2 * nslot)),
            pltpu.VMEM_SHARED((NP, dfeat), dtype),
        ],
        compiler_params=pltpu.CompilerParams(use_tc_tiling_on_sc=False),
    )


def _tc1_body(x_ref, w1_ref, deg_ref, dis_ref, hp_ref):
    deg = 1.0 + deg_ref[0] + deg_ref[1]
    dis = lax.rsqrt(deg)
    dis_ref[...] = dis
    h = jnp.dot(x_ref[...], w1_ref[...], preferred_element_type=jnp.float32)
    hp_ref[...] = (h * dis[:N]).astype(jnp.bfloat16)


def _tc2_body(p_ref, hp_ref, dis_ref, b1_ref, w2_ref, h2p_ref):
    d = dis_ref[...][:N]
    agg = (p_ref[0, pl.ds(0, N), :].astype(jnp.float32)
           + p_ref[1, pl.ds(0, N), :].astype(jnp.float32)
           + hp_ref[...].astype(jnp.float32))
    z = jnp.maximum(agg * d + b1_ref[...], 0.0)
    h2 = jnp.dot(z, w2_ref[...], preferred_element_type=jnp.float32)
    h2p_ref[...] = h2 * d


def _tc3_body(q_ref, h2p_ref, dis_ref, b2_ref, out_ref):
    d = dis_ref[...][:N]
    logits = (q_ref[0, pl.ds(0, N), :] + q_ref[1, pl.ds(0, N), :]
              + h2p_ref[...]) * d + b2_ref[...]
    m = jnp.max(logits, axis=1, keepdims=True)
    lse = jnp.log(jnp.sum(jnp.exp(logits - m), axis=1, keepdims=True)) + m
    out_ref[...] = logits - lse


def _tc_call(body, out_shapes):
    return pl.pallas_call(body, out_shape=out_shapes)


@jax.jit
def kernel(x, edge_index, W1, b1, W2, b2):
    src = edge_index[0]
    dst = edge_index[1]
    pad = EP - E
    pad_i = jnp.arange(pad, dtype=jnp.int32)
    src2d = jnp.concatenate(
        [src, pad_i % N]).reshape(IDX_ROWS, EC)
    dst2d = jnp.concatenate(
        [dst, TRASH + pad_i % (NP - TRASH)]).reshape(IDX_ROWS, EC)

    deg_part = _make_deg_kernel()(dst2d)

    dis, hp = _tc_call(_tc1_body, (
        jax.ShapeDtypeStruct((NP, 1), jnp.float32),
        jax.ShapeDtypeStruct((N, D_HID), jnp.bfloat16),
    ))(x, W1, deg_part.reshape(NC, NP, 1))

    p = _make_agg_kernel(D_HID)(hp, src2d, dst2d)
    src128 = src2d.reshape(EP // 128, 128)
    dst128 = dst2d.reshape(EP // 128, 128)

    h2p = _tc_call(_tc2_body, jax.ShapeDtypeStruct((N, D_OUT), jnp.float32))(
        p, hp, dis, b1.reshape(1, D_HID), W2)

    q = _make_agg_kernel(D_OUT)(h2p, src128, dst128)

    out = _tc_call(_tc3_body, jax.ShapeDtypeStruct((N, D_OUT), jnp.float32))(
        q, h2p, dis, b2.reshape(1, D_OUT))
    return out


# speedup vs baseline: 1.1259x; 1.0343x over previous
import functools

import jax
import jax.numpy as jnp
from jax import lax
from jax.experimental import pallas as pl
from jax.experimental.pallas import tpu as pltpu
from jax.experimental.pallas import tpu_sc as plsc

N = 10000
E = 320000
D_IN = 128
D_HID = 128
D_OUT = 16

NC = 2
NS = 16
NW = NC * NS

NP = 10240
TRASH = 10000
EP = 327680
EC = 64
IDX_ROWS = EP // EC
ROWS_PER_W = IDX_ROWS // NW
IB = 16
ZCHUNK = NP // NS


def _mesh():
    return plsc.VectorSubcoreMesh(
        core_axis_name="c", subcore_axis_name="s", num_cores=NC, num_subcores=NS
    )


def _deg_body(dst_hbm, out_hbm, idx_v, ones_v, zeros_v, sem, acc_sh):
    c = lax.axis_index("c")
    s = lax.axis_index("s")
    wid = c * NS + s

    def _z(i, _):
        zeros_v[pl.ds(i * 16, 16)] = jnp.zeros((16,), jnp.float32)
        return _
    lax.fori_loop(0, ZCHUNK // 16, _z, None)
    for k in range(EC // 16):
        ones_v[pl.ds(k * 16, 16)] = jnp.ones((16,), jnp.float32)

    pltpu.sync_copy(zeros_v, acc_sh.at[pl.ds(s * ZCHUNK, ZCHUNK)])
    plsc.subcore_barrier()

    def _blk(b, _):
        pltpu.sync_copy(dst_hbm.at[pl.ds(wid * ROWS_PER_W + b * IB, IB)], idx_v)
        for j in range(IB):
            pltpu.async_copy(ones_v, acc_sh.at[idx_v.at[j]], sem, add=True)
        for j in range(IB):
            pltpu.make_async_copy(ones_v, acc_sh.at[idx_v.at[0]], sem).wait()
        return _
    lax.fori_loop(0, ROWS_PER_W // IB, _blk, None)

    plsc.subcore_barrier()
    pltpu.sync_copy(
        acc_sh.at[pl.ds(s * ZCHUNK, ZCHUNK)],
        out_hbm.at[c, pl.ds(s * ZCHUNK, ZCHUNK)],
    )


def _make_deg_kernel():
    return pl.kernel(
        _deg_body,
        out_type=jax.ShapeDtypeStruct((NC, NP), jnp.float32),
        mesh=_mesh(),
        scratch_types=[
            pltpu.VMEM((IB, EC), jnp.int32),
            pltpu.VMEM((EC,), jnp.float32),
            pltpu.VMEM((ZCHUNK,), jnp.float32),
            pltpu.SemaphoreType.DMA,
            pltpu.VMEM_SHARED((NP,), jnp.float32),
        ],
    )


def _agg_body(h_hbm, src_hbm, dst_hbm, out_hbm,
              src_v, dst_v, rows_v, *rest, dfeat, nslot, ahead, ec, ib,
              rows_per_w, dtype):
    sg = rest[:nslot]
    ss = rest[nslot:2 * nslot]
    acc_sh = rest[2 * nslot]
    c = lax.axis_index("c")
    s = lax.axis_index("s")
    wid = c * NS + s
    lanes = 16 if dtype == jnp.float32 else 32
    nvec = dfeat // lanes

    def _z(r, _):
        for k in range(nvec):
            rows_v[0, r, pl.ds(k * lanes, lanes)] = jnp.zeros((lanes,), dtype)
        return _
    lax.fori_loop(0, ec, _z, None)
    for i in range(ZCHUNK // ec):
        pltpu.sync_copy(rows_v.at[0],
                        acc_sh.at[pl.ds(s * ZCHUNK + i * ec, ec)])
    plsc.subcore_barrier()

    def _wait_scatter(slot):
        pltpu.make_async_copy(
            rows_v.at[slot], acc_sh.at[dst_v.at[0]], ss[slot]).wait()

    def _blk(b, _):
        @pl.when(b > 0)
        def _():
            for slot in range(nslot):
                _wait_scatter(slot)
        base = wid * rows_per_w + b * ib
        pltpu.sync_copy(src_hbm.at[pl.ds(base, ib)], src_v)
        pltpu.sync_copy(dst_hbm.at[pl.ds(base, ib)], dst_v)

        def _fire(j, slot):
            pltpu.async_copy(h_hbm.at[src_v.at[j]], rows_v.at[slot], sg[slot])

        for k in range(ahead):
            _fire(k, k % nslot)
        for j in range(ib):
            sj = j % nslot
            if j + ahead < ib:
                s2 = (j + ahead) % nslot
                if j + ahead >= nslot:
                    _wait_scatter(s2)
                _fire(j + ahead, s2)
            pltpu.make_async_copy(
                h_hbm.at[src_v.at[j]], rows_v.at[sj], sg[sj]).wait()
            pltpu.async_copy(rows_v.at[sj], acc_sh.at[dst_v.at[j]], ss[sj],
                             add=True)
        return _
    lax.fori_loop(0, rows_per_w // ib, _blk, None)
    for slot in range(nslot):
        _wait_scatter(slot)

    plsc.subcore_barrier()
    for i in range(ZCHUNK // ec):
        pltpu.sync_copy(acc_sh.at[pl.ds(s * ZCHUNK + i * ec, ec)],
                        out_hbm.at[c, pl.ds(s * ZCHUNK + i * ec, ec)])


def _make_agg_kernel(dfeat):
    if dfeat == 128:
        nslot, ahead, ec, ib = 4, 2, 128, 16
        dtype = jnp.bfloat16
    else:
        nslot, ahead, ec, ib = 6, 3, 128, 16
        dtype = jnp.float32
    rows_per_w = EP // ec // NW
    assert EP % (ec * NW) == 0 and rows_per_w % ib == 0 and ZCHUNK % ec == 0
    return pl.kernel(
        functools.partial(_agg_body, dfeat=dfeat, nslot=nslot, ahead=ahead,
                          ec=ec, ib=ib, rows_per_w=rows_per_w, dtype=dtype),
        out_type=jax.ShapeDtypeStruct((NC, NP, dfeat), dtype),
        mesh=_mesh(),
        scratch_types=[
            pltpu.VMEM((ib, ec), jnp.int32),
            pltpu.VMEM((ib, ec), jnp.int32),
            pltpu.VMEM((nslot, ec, dfeat), dtype),
            *([pltpu.SemaphoreType.DMA] * (2 * nslot)),
            pltpu.VMEM_SHARED((NP, dfeat), dtype),
        ],
        compiler_params=pltpu.CompilerParams(use_tc_tiling_on_sc=False),
    )


def _tc1_body(x_ref, w1_ref, deg_ref, dis_ref, hp_ref):
    deg = 1.0 + deg_ref[0] + deg_ref[1]
    dis = lax.rsqrt(deg)
    dis_ref[...] = dis
    h = jnp.dot(x_ref[...], w1_ref[...], preferred_element_type=jnp.float32)
    hp_ref[...] = (h * dis[:N]).astype(jnp.bfloat16)


def _tc2_body(p_ref, hp_ref, dis_ref, b1_ref, w2_ref, h2p_ref):
    d = dis_ref[...][:N]
    agg = (p_ref[0, pl.ds(0, N), :].astype(jnp.float32)
           + p_ref[1, pl.ds(0, N), :].astype(jnp.float32)
           + hp_ref[...].astype(jnp.float32))
    z = jnp.maximum(agg * d + b1_ref[...], 0.0)
    h2 = jnp.dot(z, w2_ref[...], preferred_element_type=jnp.float32)
    h2p_ref[...] = h2 * d


def _tc3_body(q_ref, h2p_ref, dis_ref, b2_ref, out_ref):
    d = dis_ref[...][:N]
    logits = (q_ref[0, pl.ds(0, N), :] + q_ref[1, pl.ds(0, N), :]
              + h2p_ref[...]) * d + b2_ref[...]
    m = jnp.max(logits, axis=1, keepdims=True)
    lse = jnp.log(jnp.sum(jnp.exp(logits - m), axis=1, keepdims=True)) + m
    out_ref[...] = logits - lse


def _tc_call(body, out_shapes):
    return pl.pallas_call(body, out_shape=out_shapes)


@jax.jit
def kernel(x, edge_index, W1, b1, W2, b2):
    src = edge_index[0]
    dst = edge_index[1]
    pad = EP - E
    pad_i = jnp.arange(pad, dtype=jnp.int32)
    src2d = jnp.concatenate(
        [src, pad_i % N]).reshape(IDX_ROWS, EC)
    dst2d = jnp.concatenate(
        [dst, TRASH + pad_i % (NP - TRASH)]).reshape(IDX_ROWS, EC)

    deg_part = _make_deg_kernel()(dst2d)

    dis, hp = _tc_call(_tc1_body, (
        jax.ShapeDtypeStruct((NP, 1), jnp.float32),
        jax.ShapeDtypeStruct((N, D_HID), jnp.bfloat16),
    ))(x, W1, deg_part.reshape(NC, NP, 1))

    src128 = src2d.reshape(EP // 128, 128)
    dst128 = dst2d.reshape(EP // 128, 128)
    p = _make_agg_kernel(D_HID)(hp, src128, dst128)

    h2p = _tc_call(_tc2_body, jax.ShapeDtypeStruct((N, D_OUT), jnp.float32))(
        p, hp, dis, b1.reshape(1, D_HID), W2)

    q = _make_agg_kernel(D_OUT)(h2p, src128, dst128)

    out = _tc_call(_tc3_body, jax.ShapeDtypeStruct((N, D_OUT), jnp.float32))(
        q, h2p, dis, b2.reshape(1, D_OUT))
    return out
